# Initial kernel scaffold; baseline (speedup 1.0000x reference)
import jax
import jax.numpy as jnp
from jax.experimental import pallas as pl


def kernel(z_sub, belong, W_out, b_out):
    raise NotImplementedError("write your pallas kernel here")


# speedup vs baseline: 1.0549x; 1.0549x over previous
import functools

import jax
import jax.numpy as jnp
from jax import lax
from jax.experimental import pallas as pl
from jax.experimental.pallas import tpu as pltpu
from jax.experimental.pallas import tpu_sc as plsc

N = 320000
D = 128
S = 10000
SPH = 5120
SP = 2 * SPH
NC = 2
NS = 16
RPT = N // NS
CH = 32
NCHUNK = RPT // CH
SEG_PER_TILE = SPH // NS
ZCH = 64

_mesh = plsc.VectorSubcoreMesh(core_axis_name="c", subcore_axis_name="s",
                               num_cores=2)


def _remap(idxbuf, idxadj, c):
    base = c * SPH
    for l in range(CH // 16):
        iv = idxbuf[pl.ds(l * 16, 16)]
        loc = iv - base
        ok = jnp.logical_and(loc >= 0, loc < SPH)
        idxadj[pl.ds(l * 16, 16)] = jnp.where(ok, loc, SPH)


@functools.partial(
    pl.kernel,
    out_type=jax.ShapeDtypeStruct((NC, SPH, D), jnp.float32),
    mesh=_mesh,
    scratch_types=[
        pltpu.VMEM((CH, D), jnp.float32),
        pltpu.VMEM((CH,), jnp.int32),
        pltpu.VMEM((CH,), jnp.int32),
        pltpu.VMEM_SHARED((SPH + 8, D), jnp.float32),
    ],
)
def _sc_segment_sum(z_hbm, idx_hbm, zeros_hbm, sums_out, zbuf, idxbuf, idxadj,
                    acc):
    c = lax.axis_index("c")
    s = lax.axis_index("s")
    row0 = s * RPT

    for k in range(SEG_PER_TILE // ZCH):
        base = s * SEG_PER_TILE + k * ZCH
        pltpu.sync_copy(zeros_hbm, acc.at[pl.ds(base, ZCH)])
    plsc.subcore_barrier()

    def _chunk(j, _):
        pltpu.sync_copy(idx_hbm.at[pl.ds(row0 + j * CH, CH)], idxbuf)
        pltpu.sync_copy(z_hbm.at[pl.ds(row0 + j * CH, CH)], zbuf)
        _remap(idxbuf, idxadj, c)
        pltpu.sync_copy(zbuf, acc.at[idxadj], add=True)
        return 0

    lax.fori_loop(0, NCHUNK, _chunk, 0)
    plsc.subcore_barrier()

    sl = pl.ds(s * SEG_PER_TILE, SEG_PER_TILE)
    pltpu.sync_copy(acc.at[sl], sums_out.at[c, sl])


@functools.partial(
    pl.kernel,
    out_type=jax.ShapeDtypeStruct((NC, SPH, D), jnp.float32),
    mesh=_mesh,
    scratch_types=[
        pltpu.VMEM((CH, D), jnp.float32),
        pltpu.VMEM((CH,), jnp.int32),
        pltpu.VMEM((CH,), jnp.int32),
        pltpu.VMEM_SHARED((SPH + 8, D), jnp.float32),
    ],
)
def _sc_segment_count(idx_hbm, zeros_hbm, ones_hbm, cnts_out, onesbuf, idxbuf,
                      idxadj, acc):
    c = lax.axis_index("c")
    s = lax.axis_index("s")
    row0 = s * RPT

    pltpu.sync_copy(ones_hbm, onesbuf)
    for k in range(SEG_PER_TILE // ZCH):
        base = s * SEG_PER_TILE + k * ZCH
        pltpu.sync_copy(zeros_hbm, acc.at[pl.ds(base, ZCH)])
    plsc.subcore_barrier()

    def _chunk(j, _):
        pltpu.sync_copy(idx_hbm.at[pl.ds(row0 + j * CH, CH)], idxbuf)
        _remap(idxbuf, idxadj, c)
        pltpu.sync_copy(onesbuf, acc.at[idxadj], add=True)
        return 0

    lax.fori_loop(0, NCHUNK, _chunk, 0)
    plsc.subcore_barrier()

    sl = pl.ds(s * SEG_PER_TILE, SEG_PER_TILE)
    pltpu.sync_copy(acc.at[sl], cnts_out.at[c, sl])


def _finish_body(sums_ref, cnt_ref, w_ref, b_ref, o_ref):
    total = jnp.concatenate([sums_ref[0], sums_ref[1]], axis=0)
    counts = jnp.concatenate([cnt_ref[0][:, :1], cnt_ref[1][:, :1]], axis=0)
    inv = 1.0 / jnp.maximum(counts, 1.0)
    o_ref[...] = (
        jnp.dot(total * inv, w_ref[...], preferred_element_type=jnp.float32)
        + b_ref[...]
    )


def kernel(z_sub, belong, W_out, b_out):
    idx = belong.astype(jnp.int32)
    zeros = jnp.zeros((ZCH, D), jnp.float32)
    ones = jnp.ones((CH, D), jnp.float32)
    sums = _sc_segment_sum(z_sub, idx, zeros)
    cnts = _sc_segment_count(idx, zeros, ones)
    out = pl.pallas_call(
        _finish_body,
        out_shape=jax.ShapeDtypeStruct((SP, W_out.shape[1]), jnp.float32),
    )(sums, cnts, W_out, b_out.reshape(1, -1))
    return out[:S]


# speedup vs baseline: 1.8271x; 1.7319x over previous
import functools

import jax
import jax.numpy as jnp
from jax import lax
from jax.experimental import pallas as pl
from jax.experimental.pallas import tpu as pltpu
from jax.experimental.pallas import tpu_sc as plsc

N = 320000
D = 128
S = 10000
SPH = 5120
SP = 2 * SPH
NC = 2
NS = 16
RPT = N // NS
CH = 80
NCHUNK = RPT // CH
SEG_PER_TILE = SPH // NS
ZCH = 64

_mesh = plsc.VectorSubcoreMesh(core_axis_name="c", subcore_axis_name="s",
                               num_cores=2)


def _remap(idxbuf, idxadj, c):
    base = c * SPH
    for l in range(CH // 16):
        iv = idxbuf[pl.ds(l * 16, 16)]
        loc = iv - base
        ok = jnp.logical_and(loc >= 0, loc < SPH)
        idxadj[pl.ds(l * 16, 16)] = jnp.where(ok, loc, SPH)


@functools.partial(
    pl.kernel,
    out_type=jax.ShapeDtypeStruct((NC, SPH, D), jnp.float32),
    mesh=_mesh,
    scratch_types=[
        pltpu.VMEM((CH, D), jnp.float32),
        pltpu.VMEM((CH,), jnp.int32),
        pltpu.VMEM((CH,), jnp.int32),
        pltpu.VMEM_SHARED((SPH + 8, D), jnp.float32),
    ],
)
def _sc_segment_sum(z_hbm, idx_hbm, zeros_hbm, sums_out, zbuf, idxbuf, idxadj,
                    acc):
    c = lax.axis_index("c")
    s = lax.axis_index("s")
    row0 = s * RPT

    for k in range(SEG_PER_TILE // ZCH):
        base = s * SEG_PER_TILE + k * ZCH
        pltpu.sync_copy(zeros_hbm, acc.at[pl.ds(base, ZCH)])
    plsc.subcore_barrier()

    def _chunk(j, _):
        pltpu.sync_copy(idx_hbm.at[pl.ds(row0 + j * CH, CH)], idxbuf)
        pltpu.sync_copy(z_hbm.at[pl.ds(row0 + j * CH, CH)], zbuf)
        _remap(idxbuf, idxadj, c)
        pltpu.sync_copy(zbuf, acc.at[idxadj], add=True)
        return 0

    lax.fori_loop(0, NCHUNK, _chunk, 0)
    plsc.subcore_barrier()

    sl = pl.ds(s * SEG_PER_TILE, SEG_PER_TILE)
    pltpu.sync_copy(acc.at[sl], sums_out.at[c, sl])


@functools.partial(
    pl.kernel,
    out_type=jax.ShapeDtypeStruct((NC, SPH, D), jnp.float32),
    mesh=_mesh,
    scratch_types=[
        pltpu.VMEM((CH, D), jnp.float32),
        pltpu.VMEM((CH,), jnp.int32),
        pltpu.VMEM((CH,), jnp.int32),
        pltpu.VMEM_SHARED((SPH + 8, D), jnp.float32),
    ],
)
def _sc_segment_count(idx_hbm, zeros_hbm, ones_hbm, cnts_out, onesbuf, idxbuf,
                      idxadj, acc):
    c = lax.axis_index("c")
    s = lax.axis_index("s")
    row0 = s * RPT

    pltpu.sync_copy(ones_hbm, onesbuf)
    for k in range(SEG_PER_TILE // ZCH):
        base = s * SEG_PER_TILE + k * ZCH
        pltpu.sync_copy(zeros_hbm, acc.at[pl.ds(base, ZCH)])
    plsc.subcore_barrier()

    def _chunk(j, _):
        pltpu.sync_copy(idx_hbm.at[pl.ds(row0 + j * CH, CH)], idxbuf)
        _remap(idxbuf, idxadj, c)
        pltpu.sync_copy(onesbuf, acc.at[idxadj], add=True)
        return 0

    lax.fori_loop(0, NCHUNK, _chunk, 0)
    plsc.subcore_barrier()

    sl = pl.ds(s * SEG_PER_TILE, SEG_PER_TILE)
    pltpu.sync_copy(acc.at[sl], cnts_out.at[c, sl])


def _finish_body(sums_ref, cnt_ref, w_ref, b_ref, o_ref):
    total = jnp.concatenate([sums_ref[0], sums_ref[1]], axis=0)
    counts = jnp.concatenate([cnt_ref[0][:, :1], cnt_ref[1][:, :1]], axis=0)
    inv = 1.0 / jnp.maximum(counts, 1.0)
    o_ref[...] = (
        jnp.dot(total * inv, w_ref[...], preferred_element_type=jnp.float32)
        + b_ref[...]
    )


def kernel(z_sub, belong, W_out, b_out):
    idx = belong.astype(jnp.int32)
    zeros = jnp.zeros((ZCH, D), jnp.float32)
    ones = jnp.ones((CH, D), jnp.float32)
    sums = _sc_segment_sum(z_sub, idx, zeros)
    cnts = _sc_segment_count(idx, zeros, ones)
    out = pl.pallas_call(
        _finish_body,
        out_shape=jax.ShapeDtypeStruct((SP, W_out.shape[1]), jnp.float32),
    )(sums, cnts, W_out, b_out.reshape(1, -1))
    return out[:S]
